# baseline (speedup 1.0000x reference)
import jax
import jax.numpy as jnp
from jax import lax
from jax.experimental import pallas as pl
from jax.experimental.pallas import tpu as pltpu

N = 100000
HIDDEN = 128
ATTN = 128
B = 512
SUB = 500
NSUBBLK = 8
BLK = SUB * NSUBBLK
NBLK = N // BLK
NSUB = NBLK * NSUBBLK
SW = 32
SWMAX = SW - 7
B8 = B // 8
SW8 = SW // 8


def _make_body(full):
    def _body(base8_ref, x_ref, b3_ref, w1_ref, b1_ref, w2_ref,
              out_ref, d_scr):
        pid = pl.program_id(0)

        @pl.when(pid == 0)
        def _():
            d_scr[...] = jnp.zeros((B8, 8, 8), jnp.float32)
            out_ref[...] = jnp.zeros((B8, 8, HIDDEN), jnp.float32)

        w1 = w1_ref[...]
        w2 = w2_ref[...]
        b1 = b1_ref[...]
        ones = jnp.ones((SUB, 8), jnp.bfloat16)
        m0 = jnp.sum(jnp.abs(w2.astype(jnp.float32)))

        for j in range(NSUBBLK):
            xb = x_ref[j * SUB:(j + 1) * SUB, :].astype(jnp.bfloat16)
            h = jnp.tanh(
                lax.dot_general(xb, w1, (((1,), (0,)), ((), ())),
                                preferred_element_type=jnp.float32)
                + b1
            )
            s_row = lax.dot_general(w2, h.astype(jnp.bfloat16),
                                    (((1,), (1,)), ((), ())),
                                    preferred_element_type=jnp.float32)

            b_row = b3_ref[0, j, :].reshape(1, SUB)
            e = jnp.exp(s_row - m0)

            if full:
                nwin, nwin8 = B, B8
                seg_col = lax.broadcasted_iota(jnp.int32, (B, 1), 0)
                d_sl = d_scr.at[...]
                o_sl = out_ref.at[...]
            else:
                nwin, nwin8 = SW, SW8
                cb8 = base8_ref[pid * NSUBBLK + j]
                seg_col = cb8 * 8 + lax.broadcasted_iota(jnp.int32, (SW, 1), 0)
                d_sl = d_scr.at[pl.ds(cb8, SW8)]
                o_sl = out_ref.at[pl.ds(cb8, SW8)]

            a = jnp.where(seg_col == b_row, e, 0.0).astype(jnp.bfloat16)
            d_sl[...] += lax.dot_general(
                a, ones, (((1,), (0,)), ((), ())),
                preferred_element_type=jnp.float32).reshape(nwin8, 8, 8)
            o_sl[...] += lax.dot_general(
                a, xb, (((1,), (0,)), ((), ())),
                preferred_element_type=jnp.float32).reshape(nwin8, 8, HIDDEN)

        @pl.when(pid == pl.num_programs(0) - 1)
        def _():
            out_ref[...] = out_ref[...] / (d_scr[:, :, 0:1] + 1e-16)

    return _body


def _call(full, bases8, x, b3, w1b, b1r, w2b):
    out = pl.pallas_call(
        _make_body(full),
        grid=(NBLK,),
        in_specs=[
            pl.BlockSpec(memory_space=pltpu.SMEM),
            pl.BlockSpec((BLK, HIDDEN), lambda i: (i, 0)),
            pl.BlockSpec((1, NSUBBLK, SUB), lambda i: (i, 0, 0)),
            pl.BlockSpec((HIDDEN, ATTN), lambda i: (0, 0)),
            pl.BlockSpec((1, ATTN), lambda i: (0, 0)),
            pl.BlockSpec((1, ATTN), lambda i: (0, 0)),
        ],
        out_specs=pl.BlockSpec((B8, 8, HIDDEN), lambda i: (0, 0, 0)),
        out_shape=jax.ShapeDtypeStruct((B8, 8, HIDDEN), jnp.float32),
        scratch_shapes=[pltpu.VMEM((B8, 8, 8), jnp.float32)],
    )(bases8, x, b3, w1b, b1r, w2b)
    return out


def kernel(x, batch, W1, b1, W2, b2):
    del b2
    bi = batch.astype(jnp.int32)
    b3 = bi.reshape(NBLK, NSUBBLK, SUB)
    idx = jnp.arange(NSUB, dtype=jnp.int32)
    bases = bi[idx * SUB]
    widths = bi[(idx + 1) * SUB - 1] - bases + 1
    bases8 = jnp.minimum(bases >> 3, B8 - SW8)
    b1r = b1.reshape(1, HIDDEN)
    w1b = W1.astype(jnp.bfloat16)
    w2b = W2.reshape(1, ATTN).astype(jnp.bfloat16)

    pooled = lax.cond(
        jnp.max(widths) <= SWMAX,
        lambda ops: _call(False, *ops),
        lambda ops: _call(True, *ops),
        (bases8, x, b3, w1b, b1r, w2b),
    )
    return pooled.reshape(B, HIDDEN)


# speedup vs baseline: 1.2754x; 1.2754x over previous
import jax
import jax.numpy as jnp
from jax import lax
from jax.experimental import pallas as pl
from jax.experimental.pallas import tpu as pltpu

N = 100000
HIDDEN = 128
ATTN = 128
B = 512
SUB = 1000
NSUBBLK = 4
BLK = SUB * NSUBBLK
NBLK = N // BLK
NSUB = NBLK * NSUBBLK
SW = 32
SWMAX = SW - 7
B8 = B // 8
SW8 = SW // 8


def _make_body(full):
    def _body(base8_ref, x_ref, b3_ref, w1_ref, b1_ref, w2_ref,
              out_ref, d_scr):
        pid = pl.program_id(0)

        @pl.when(pid == 0)
        def _():
            d_scr[...] = jnp.zeros((B8, 8, 8), jnp.float32)
            out_ref[...] = jnp.zeros((B8, 8, HIDDEN), jnp.float32)

        w1 = w1_ref[...]
        w2 = w2_ref[...]
        b1 = b1_ref[...]
        ones = jnp.ones((SUB, 8), jnp.bfloat16)
        m0 = jnp.sum(jnp.abs(w2.astype(jnp.float32)))

        for j in range(NSUBBLK):
            xb = x_ref[j * SUB:(j + 1) * SUB, :].astype(jnp.bfloat16)
            h = jnp.tanh(
                lax.dot_general(xb, w1, (((1,), (0,)), ((), ())),
                                preferred_element_type=jnp.float32)
                + b1
            )
            s_row = lax.dot_general(w2, h.astype(jnp.bfloat16),
                                    (((1,), (1,)), ((), ())),
                                    preferred_element_type=jnp.float32)

            b_row = b3_ref[0, j, :].reshape(1, SUB)
            e = jnp.exp(s_row - m0)

            if full:
                nwin, nwin8 = B, B8
                seg_col = lax.broadcasted_iota(jnp.int32, (B, 1), 0)
                d_sl = d_scr.at[...]
                o_sl = out_ref.at[...]
            else:
                nwin, nwin8 = SW, SW8
                cb8 = base8_ref[pid * NSUBBLK + j]
                seg_col = cb8 * 8 + lax.broadcasted_iota(jnp.int32, (SW, 1), 0)
                d_sl = d_scr.at[pl.ds(cb8, SW8)]
                o_sl = out_ref.at[pl.ds(cb8, SW8)]

            a = jnp.where(seg_col == b_row, e, 0.0).astype(jnp.bfloat16)
            d_sl[...] += lax.dot_general(
                a, ones, (((1,), (0,)), ((), ())),
                preferred_element_type=jnp.float32).reshape(nwin8, 8, 8)
            o_sl[...] += lax.dot_general(
                a, xb, (((1,), (0,)), ((), ())),
                preferred_element_type=jnp.float32).reshape(nwin8, 8, HIDDEN)

        @pl.when(pid == pl.num_programs(0) - 1)
        def _():
            out_ref[...] = out_ref[...] / (d_scr[:, :, 0:1] + 1e-16)

    return _body


def _call(full, bases8, x, b3, w1b, b1r, w2b):
    out = pl.pallas_call(
        _make_body(full),
        grid=(NBLK,),
        in_specs=[
            pl.BlockSpec(memory_space=pltpu.SMEM),
            pl.BlockSpec((BLK, HIDDEN), lambda i: (i, 0)),
            pl.BlockSpec((1, NSUBBLK, SUB), lambda i: (i, 0, 0)),
            pl.BlockSpec((HIDDEN, ATTN), lambda i: (0, 0)),
            pl.BlockSpec((1, ATTN), lambda i: (0, 0)),
            pl.BlockSpec((1, ATTN), lambda i: (0, 0)),
        ],
        out_specs=pl.BlockSpec((B8, 8, HIDDEN), lambda i: (0, 0, 0)),
        out_shape=jax.ShapeDtypeStruct((B8, 8, HIDDEN), jnp.float32),
        scratch_shapes=[pltpu.VMEM((B8, 8, 8), jnp.float32)],
    )(bases8, x, b3, w1b, b1r, w2b)
    return out


def kernel(x, batch, W1, b1, W2, b2):
    del b2
    bi = batch.astype(jnp.int32)
    b3 = bi.reshape(NBLK, NSUBBLK, SUB)
    idx = jnp.arange(NSUB, dtype=jnp.int32)
    bases = bi[idx * SUB]
    widths = bi[(idx + 1) * SUB - 1] - bases + 1
    bases8 = jnp.minimum(bases >> 3, B8 - SW8)
    b1r = b1.reshape(1, HIDDEN)
    w1b = W1.astype(jnp.bfloat16)
    w2b = W2.reshape(1, ATTN).astype(jnp.bfloat16)

    pooled = lax.cond(
        jnp.max(widths) <= SWMAX,
        lambda ops: _call(False, *ops),
        lambda ops: _call(True, *ops),
        (bases8, x, b3, w1b, b1r, w2b),
    )
    return pooled.reshape(B, HIDDEN)
